# initial kernel scaffold (unmeasured)
import jax
import jax.numpy as jnp
from jax import lax
from jax.experimental import pallas as pl
from jax.experimental.pallas import tpu as pltpu

N_DEV = 4


def kernel(x, w_mat):
    m_per, k = x.shape
    _, n_per = w_mat.shape

    def body(x_ref, w_ref, out_ref, comm_ref, amax_ref,
             send_sems, recv_sems, a_send_sems, a_recv_sems):
        my = lax.axis_index("i")
        left = lax.rem(my + N_DEV - 1, N_DEV)
        right = lax.rem(my + 1, N_DEV)

        barrier_sem = pltpu.get_barrier_semaphore()
        for nbr in (left, right):
            pl.semaphore_signal(
                barrier_sem, inc=1,
                device_id=(nbr,), device_id_type=pl.DeviceIdType.MESH,
            )
        pl.semaphore_wait(barrier_sem, 2)

        def gemm_store(src, origin):
            y = lax.dot_general(
                src, w_ref[...],
                dimension_numbers=(((1,), (0,)), ((), ())),
                precision=lax.Precision.HIGHEST,
                preferred_element_type=jnp.float32,
            )
            out_ref[pl.ds(origin * m_per, m_per), :] = jnp.maximum(y, 0.0)

        gemm_store(x_ref[...], my)

        for h in range(N_DEV - 1):
            src = x_ref if h == 0 else comm_ref.at[h - 1]
            rdma = pltpu.make_async_remote_copy(
                src_ref=src,
                dst_ref=comm_ref.at[h],
                send_sem=send_sems.at[h],
                recv_sem=recv_sems.at[h],
                device_id=(right,),
                device_id_type=pl.DeviceIdType.MESH,
            )
            rdma.start()
            rdma.wait()
            origin = lax.rem(my + 2 * N_DEV - 1 - h, N_DEV)
            gemm_store(comm_ref[h], origin)

        amax_local = jnp.max(out_ref[...])
        amax_ref[pl.ds(my, 1), :, :] = jnp.full(
            (1, 8, 128), amax_local, jnp.float32)

        rdmas = []
        for o in range(1, N_DEV):
            tgt = lax.rem(my + o, N_DEV)
            r = pltpu.make_async_remote_copy(
                src_ref=amax_ref.at[pl.ds(my, 1)],
                dst_ref=amax_ref.at[pl.ds(my, 1)],
                send_sem=a_send_sems.at[o - 1],
                recv_sem=a_recv_sems.at[o - 1],
                device_id=(tgt,),
                device_id_type=pl.DeviceIdType.MESH,
            )
            r.start()
            rdmas.append(r)
        for r in rdmas:
            r.wait()

        g_amax = jnp.max(amax_ref[...])
        scale = g_amax / 448.0
        q = (out_ref[...] / scale).astype(jnp.float8_e4m3fn)
        out_ref[...] = q.astype(jnp.float32) * scale

    return pl.pallas_call(
        body,
        out_shape=jax.ShapeDtypeStruct((N_DEV * m_per, n_per), jnp.float32),
        in_specs=[
            pl.BlockSpec(memory_space=pltpu.VMEM),
            pl.BlockSpec(memory_space=pltpu.VMEM),
        ],
        out_specs=pl.BlockSpec(memory_space=pltpu.VMEM),
        scratch_shapes=[
            pltpu.VMEM((N_DEV - 1, m_per, k), jnp.float32),
            pltpu.VMEM((N_DEV, 8, 128), jnp.float32),
            pltpu.SemaphoreType.DMA((N_DEV - 1,)),
            pltpu.SemaphoreType.DMA((N_DEV - 1,)),
            pltpu.SemaphoreType.DMA((N_DEV - 1,)),
            pltpu.SemaphoreType.DMA((N_DEV - 1,)),
        ],
        compiler_params=pltpu.CompilerParams(collective_id=0),
    )(x, w_mat)


# baseline (device time: 711245 ns/iter reference)
import jax
import jax.numpy as jnp
from jax import lax
from jax.experimental import pallas as pl
from jax.experimental.pallas import tpu as pltpu

N_DEV = 4
HALF = 4


def kernel(x, w_mat):
    m_per, k = x.shape
    _, n_per = w_mat.shape
    m_half = m_per // HALF

    def body(x_ref, w_ref, out_ref, comm_ref, stage_ref, amax_ref,
             send_sems, recv_sems, a_send_sems, a_recv_sems, copy_sems):
        my = lax.axis_index("i")
        left = lax.rem(my + N_DEV - 1, N_DEV)
        right = lax.rem(my + 1, N_DEV)

        barrier_sem = pltpu.get_barrier_semaphore()
        for nbr in (left, right):
            pl.semaphore_signal(
                barrier_sem, inc=1,
                device_id=(nbr,), device_id_type=pl.DeviceIdType.MESH,
            )
        pl.semaphore_wait(barrier_sem, 2)

        def gemm_store(src_val, row_start):
            y = lax.dot_general(
                src_val, w_ref[...],
                dimension_numbers=(((1,), (0,)), ((), ())),
                precision=lax.Precision.HIGHEST,
                preferred_element_type=jnp.float32,
            )
            out_ref[pl.ds(row_start, m_half), :] = jnp.maximum(y, 0.0)

        for j in range(HALF):
            gemm_store(x_ref[pl.ds(j * m_half, m_half), :],
                       my * m_per + j * m_half)

        for h in range(N_DEV - 1):
            src = x_ref if h == 0 else comm_ref.at[h - 1]
            rdma = pltpu.make_async_remote_copy(
                src_ref=src,
                dst_ref=comm_ref.at[h],
                send_sem=send_sems.at[h],
                recv_sem=recv_sems.at[h],
                device_id=(right,),
                device_id_type=pl.DeviceIdType.MESH,
            )
            rdma.start()
            rdma.wait()

            origin = lax.rem(my + 2 * N_DEV - 1 - h, N_DEV)
            for j in range(HALF):
                slot = (h * HALF + j) % 2
                cp = pltpu.make_async_copy(
                    comm_ref.at[h].at[pl.ds(j * m_half, m_half), :],
                    stage_ref.at[slot],
                    copy_sems.at[slot],
                )
                cp.start()
                cp.wait()
                gemm_store(stage_ref[slot], origin * m_per + j * m_half)

        amax_local = jnp.float32(0)
        for b in range(N_DEV):
            amax_local = jnp.maximum(
                amax_local,
                jnp.max(out_ref[pl.ds(b * m_per, m_per), :]))
        amax_ref[pl.ds(my, 1), :, :] = jnp.full(
            (1, 8, 128), amax_local, jnp.float32)

        rdmas = []
        for o in range(1, N_DEV):
            tgt = lax.rem(my + o, N_DEV)
            r = pltpu.make_async_remote_copy(
                src_ref=amax_ref.at[pl.ds(my, 1)],
                dst_ref=amax_ref.at[pl.ds(my, 1)],
                send_sem=a_send_sems.at[o - 1],
                recv_sem=a_recv_sems.at[o - 1],
                device_id=(tgt,),
                device_id_type=pl.DeviceIdType.MESH,
            )
            r.start()
            rdmas.append(r)
        for r in rdmas:
            r.wait()

        g_amax = jnp.max(amax_ref[...])
        scale = g_amax / 448.0
        inv_scale = 448.0 / g_amax
        for b in range(N_DEV):
            blk = out_ref[pl.ds(b * m_per, m_per), :]
            q = (blk * inv_scale).astype(jnp.float8_e4m3fn)
            out_ref[pl.ds(b * m_per, m_per), :] = (
                q.astype(jnp.float32) * scale)

    out, _ = pl.pallas_call(
        body,
        out_shape=[
            jax.ShapeDtypeStruct((N_DEV * m_per, n_per), jnp.float32),
            jax.ShapeDtypeStruct((N_DEV - 1, m_per, k), jnp.float32),
        ],
        in_specs=[
            pl.BlockSpec(memory_space=pltpu.VMEM),
            pl.BlockSpec(memory_space=pltpu.VMEM),
        ],
        out_specs=[
            pl.BlockSpec(memory_space=pltpu.VMEM),
            pl.BlockSpec(memory_space=pltpu.HBM),
        ],
        scratch_shapes=[
            pltpu.VMEM((2, m_half, k), jnp.float32),
            pltpu.VMEM((N_DEV, 8, 128), jnp.float32),
            pltpu.SemaphoreType.DMA((N_DEV - 1,)),
            pltpu.SemaphoreType.DMA((N_DEV - 1,)),
            pltpu.SemaphoreType.DMA((N_DEV - 1,)),
            pltpu.SemaphoreType.DMA((N_DEV - 1,)),
            pltpu.SemaphoreType.DMA((2,)),
        ],
        compiler_params=pltpu.CompilerParams(collective_id=0),
    )(x, w_mat)
    return out


# device time: 328870 ns/iter; 2.1627x vs baseline; 2.1627x over previous
import jax
import jax.numpy as jnp
from jax import lax
from jax.experimental import pallas as pl
from jax.experimental.pallas import tpu as pltpu

N_DEV = 4
QROWS = 256


def kernel(x, w_mat):
    m_per, k = x.shape
    _, n_per = w_mat.shape
    m2 = m_per // 2

    def body(x_ref, w_ref, out_ref, comm_ref, stage_ref, amax_ref,
             cw_send, cw_recv, ccw_send, ccw_recv,
             a_send, a_recv, copy_sems):
        my = lax.axis_index("i")
        left = lax.rem(my + N_DEV - 1, N_DEV)
        right = lax.rem(my + 1, N_DEV)

        barrier_sem = pltpu.get_barrier_semaphore()
        for nbr in (left, right):
            pl.semaphore_signal(
                barrier_sem, inc=1,
                device_id=(nbr,), device_id_type=pl.DeviceIdType.MESH,
            )
        pl.semaphore_wait(barrier_sem, 2)

        def gemm(src_val, row_start, amax):
            y = lax.dot_general(
                src_val, w_ref[...],
                dimension_numbers=(((1,), (0,)), ((), ())),
                precision=lax.Precision.HIGHEST,
                preferred_element_type=jnp.float32,
            )
            y = jnp.maximum(y, 0.0)
            out_ref[pl.ds(row_start, y.shape[0]), :] = y
            return jnp.maximum(amax, jnp.max(y))

        def process_hop(h, amax):
            o_cw = lax.rem(my + 2 * N_DEV - 1 - h, N_DEV)
            o_ccw = lax.rem(my + 1 + h, N_DEV)
            tasks = []
            for origin, base in ((o_cw, 0), (o_ccw, m2)):
                for j in range(m2 // QROWS):
                    r0 = base + j * QROWS
                    tasks.append((r0, origin * m_per + r0))
            cps = [
                pltpu.make_async_copy(
                    comm_ref.at[h, pl.ds(r0, QROWS), :],
                    stage_ref.at[q % 2],
                    copy_sems.at[q % 2],
                )
                for q, (r0, _) in enumerate(tasks)
            ]
            cps[0].start()
            for q, (_, out_row) in enumerate(tasks):
                cps[q].wait()
                if q + 1 < len(cps):
                    cps[q + 1].start()
                amax = gemm(stage_ref[q % 2], out_row, amax)
            return amax

        amax = jnp.float32(0)

        for h in range(N_DEV - 1):
            src = x_ref if h == 0 else comm_ref.at[h - 1]
            cw = pltpu.make_async_remote_copy(
                src_ref=src.at[pl.ds(0, m2), :],
                dst_ref=comm_ref.at[h, pl.ds(0, m2), :],
                send_sem=cw_send.at[h],
                recv_sem=cw_recv.at[h],
                device_id=(right,),
                device_id_type=pl.DeviceIdType.MESH,
            )
            ccw = pltpu.make_async_remote_copy(
                src_ref=src.at[pl.ds(m2, m2), :],
                dst_ref=comm_ref.at[h, pl.ds(m2, m2), :],
                send_sem=ccw_send.at[h],
                recv_sem=ccw_recv.at[h],
                device_id=(left,),
                device_id_type=pl.DeviceIdType.MESH,
            )
            cw.start()
            ccw.start()
            if h == 0:
                for j in range(m_per // QROWS):
                    amax = gemm(x_ref[pl.ds(j * QROWS, QROWS), :],
                                my * m_per + j * QROWS, amax)
            else:
                amax = process_hop(h - 1, amax)
            cw.wait()
            ccw.wait()
        amax = process_hop(N_DEV - 2, amax)

        amax_ref[pl.ds(my, 1), :, :] = jnp.full((1, 8, 128), amax,
                                                jnp.float32)
        rdmas = []
        for o in range(1, N_DEV):
            tgt = lax.rem(my + o, N_DEV)
            r = pltpu.make_async_remote_copy(
                src_ref=amax_ref.at[pl.ds(my, 1)],
                dst_ref=amax_ref.at[pl.ds(my, 1)],
                send_sem=a_send.at[o - 1],
                recv_sem=a_recv.at[o - 1],
                device_id=(tgt,),
                device_id_type=pl.DeviceIdType.MESH,
            )
            r.start()
            rdmas.append(r)
        for r in rdmas:
            r.wait()

        g_amax = jnp.max(amax_ref[...])
        scale = g_amax / 448.0
        inv_scale = 448.0 / g_amax
        for b in range(N_DEV):
            blk = out_ref[pl.ds(b * m_per, m_per), :]
            q = (blk * inv_scale).astype(jnp.float8_e4m3fn)
            out_ref[pl.ds(b * m_per, m_per), :] = (
                q.astype(jnp.float32) * scale)

    out, _ = pl.pallas_call(
        body,
        out_shape=[
            jax.ShapeDtypeStruct((N_DEV * m_per, n_per), jnp.float32),
            jax.ShapeDtypeStruct((N_DEV - 1, m_per, k), jnp.float32),
        ],
        in_specs=[
            pl.BlockSpec(memory_space=pltpu.VMEM),
            pl.BlockSpec(memory_space=pltpu.VMEM),
        ],
        out_specs=[
            pl.BlockSpec(memory_space=pltpu.VMEM),
            pl.BlockSpec(memory_space=pltpu.HBM),
        ],
        scratch_shapes=[
            pltpu.VMEM((2, QROWS, k), jnp.float32),
            pltpu.VMEM((N_DEV, 8, 128), jnp.float32),
            pltpu.SemaphoreType.DMA((N_DEV - 1,)),
            pltpu.SemaphoreType.DMA((N_DEV - 1,)),
            pltpu.SemaphoreType.DMA((N_DEV - 1,)),
            pltpu.SemaphoreType.DMA((N_DEV - 1,)),
            pltpu.SemaphoreType.DMA((N_DEV - 1,)),
            pltpu.SemaphoreType.DMA((N_DEV - 1,)),
            pltpu.SemaphoreType.DMA((2,)),
        ],
        compiler_params=pltpu.CompilerParams(collective_id=0),
    )(x, w_mat)
    return out


# device time: 306340 ns/iter; 2.3218x vs baseline; 1.0735x over previous
import jax
import jax.numpy as jnp
from jax import lax
from jax.experimental import pallas as pl
from jax.experimental.pallas import tpu as pltpu

N_DEV = 4
QROWS = 256


def kernel(x, w_mat):
    m_per, k = x.shape
    _, n_per = w_mat.shape
    m2 = m_per // 2

    def body(x_ref, w_ref, out_ref, comm_ref, stage_ref, amax_ref,
             cw_send, cw_recv, ccw_send, ccw_recv,
             a_send, a_recv, copy_sems):
        my = lax.axis_index("i")
        left = lax.rem(my + N_DEV - 1, N_DEV)
        right = lax.rem(my + 1, N_DEV)

        barrier_sem = pltpu.get_barrier_semaphore()
        for nbr in (left, right):
            pl.semaphore_signal(
                barrier_sem, inc=1,
                device_id=(nbr,), device_id_type=pl.DeviceIdType.MESH,
            )
        pl.semaphore_wait(barrier_sem, 2)

        def gemm(src_val, row_start, amax):
            y = lax.dot_general(
                src_val, w_ref[...],
                dimension_numbers=(((1,), (0,)), ((), ())),
                precision=lax.Precision.DEFAULT,
                preferred_element_type=jnp.float32,
            )
            y = jnp.maximum(y, 0.0)
            out_ref[pl.ds(row_start, y.shape[0]), :] = y
            return jnp.maximum(amax, jnp.max(y))

        def process_hop(h, amax):
            o_cw = lax.rem(my + 2 * N_DEV - 1 - h, N_DEV)
            o_ccw = lax.rem(my + 1 + h, N_DEV)
            tasks = []
            for origin, base in ((o_cw, 0), (o_ccw, m2)):
                for j in range(m2 // QROWS):
                    r0 = base + j * QROWS
                    tasks.append((r0, origin * m_per + r0))
            cps = [
                pltpu.make_async_copy(
                    comm_ref.at[h, pl.ds(r0, QROWS), :],
                    stage_ref.at[q % 2],
                    copy_sems.at[q % 2],
                )
                for q, (r0, _) in enumerate(tasks)
            ]
            cps[0].start()
            for q, (_, out_row) in enumerate(tasks):
                cps[q].wait()
                if q + 1 < len(cps):
                    cps[q + 1].start()
                amax = gemm(stage_ref[q % 2], out_row, amax)
            return amax

        amax = jnp.float32(0)

        for h in range(N_DEV - 1):
            src = x_ref if h == 0 else comm_ref.at[h - 1]
            cw = pltpu.make_async_remote_copy(
                src_ref=src.at[pl.ds(0, m2), :],
                dst_ref=comm_ref.at[h, pl.ds(0, m2), :],
                send_sem=cw_send.at[h],
                recv_sem=cw_recv.at[h],
                device_id=(right,),
                device_id_type=pl.DeviceIdType.MESH,
            )
            ccw = pltpu.make_async_remote_copy(
                src_ref=src.at[pl.ds(m2, m2), :],
                dst_ref=comm_ref.at[h, pl.ds(m2, m2), :],
                send_sem=ccw_send.at[h],
                recv_sem=ccw_recv.at[h],
                device_id=(left,),
                device_id_type=pl.DeviceIdType.MESH,
            )
            cw.start()
            ccw.start()
            if h == 0:
                for j in range(m_per // QROWS):
                    amax = gemm(x_ref[pl.ds(j * QROWS, QROWS), :],
                                my * m_per + j * QROWS, amax)
            else:
                amax = process_hop(h - 1, amax)
            cw.wait()
            ccw.wait()
        amax = process_hop(N_DEV - 2, amax)

        amax_ref[pl.ds(my, 1), :, :] = jnp.full((1, 8, 128), amax,
                                                jnp.float32)
        rdmas = []
        for o in range(1, N_DEV):
            tgt = lax.rem(my + o, N_DEV)
            r = pltpu.make_async_remote_copy(
                src_ref=amax_ref.at[pl.ds(my, 1)],
                dst_ref=amax_ref.at[pl.ds(my, 1)],
                send_sem=a_send.at[o - 1],
                recv_sem=a_recv.at[o - 1],
                device_id=(tgt,),
                device_id_type=pl.DeviceIdType.MESH,
            )
            r.start()
            rdmas.append(r)
        for r in rdmas:
            r.wait()

        g_amax = jnp.max(amax_ref[...])
        scale = g_amax / 448.0
        inv_scale = 448.0 / g_amax
        for b in range(N_DEV):
            blk = out_ref[pl.ds(b * m_per, m_per), :]
            q = (blk * inv_scale).astype(jnp.float8_e4m3fn)
            out_ref[pl.ds(b * m_per, m_per), :] = (
                q.astype(jnp.float32) * scale)

    out, _ = pl.pallas_call(
        body,
        out_shape=[
            jax.ShapeDtypeStruct((N_DEV * m_per, n_per), jnp.float32),
            jax.ShapeDtypeStruct((N_DEV - 1, m_per, k), jnp.float32),
        ],
        in_specs=[
            pl.BlockSpec(memory_space=pltpu.VMEM),
            pl.BlockSpec(memory_space=pltpu.VMEM),
        ],
        out_specs=[
            pl.BlockSpec(memory_space=pltpu.VMEM),
            pl.BlockSpec(memory_space=pltpu.HBM),
        ],
        scratch_shapes=[
            pltpu.VMEM((2, QROWS, k), jnp.float32),
            pltpu.VMEM((N_DEV, 8, 128), jnp.float32),
            pltpu.SemaphoreType.DMA((N_DEV - 1,)),
            pltpu.SemaphoreType.DMA((N_DEV - 1,)),
            pltpu.SemaphoreType.DMA((N_DEV - 1,)),
            pltpu.SemaphoreType.DMA((N_DEV - 1,)),
            pltpu.SemaphoreType.DMA((N_DEV - 1,)),
            pltpu.SemaphoreType.DMA((N_DEV - 1,)),
            pltpu.SemaphoreType.DMA((2,)),
        ],
        compiler_params=pltpu.CompilerParams(collective_id=0),
    )(x, w_mat)
    return out


# device time: 204583 ns/iter; 3.4766x vs baseline; 1.4974x over previous
import jax
import jax.numpy as jnp
from jax import lax
from jax.experimental import pallas as pl
from jax.experimental.pallas import tpu as pltpu

N_DEV = 4
SUB = 256


def kernel(x, w_mat):
    m_per, k = x.shape
    _, n_per = w_mat.shape
    n2 = n_per // 2

    def body(x_ref, w_ref, out_ref, wcomm_ref, wstage_ref, ysend_ref,
             amax_ref, cw_send, cw_recv, ccw_send, ccw_recv,
             y_send, y_recv, a_send, a_recv, wcopy_sems):
        my = lax.axis_index("i")
        left = lax.rem(my + N_DEV - 1, N_DEV)
        right = lax.rem(my + 1, N_DEV)

        barrier_sem = pltpu.get_barrier_semaphore()
        for nbr in (left, right):
            pl.semaphore_signal(
                barrier_sem, inc=1,
                device_id=(nbr,), device_id_type=pl.DeviceIdType.MESH,
            )
        pl.semaphore_wait(barrier_sem, 2)

        def matmul(x_rows, w_val):
            y = lax.dot_general(
                x_ref[pl.ds(x_rows, SUB), :], w_val,
                dimension_numbers=(((1,), (0,)), ((), ())),
                precision=lax.Precision.DEFAULT,
                preferred_element_type=jnp.float32,
            )
            return jnp.maximum(y, 0.0)

        y_rdmas = []

        def process_w_hop(h, amax):
            o_cw = lax.rem(my + 2 * N_DEV - 1 - h, N_DEV)
            o_ccw = lax.rem(my + 1 + h, N_DEV)
            cps = []
            for d in range(2):
                cp = pltpu.make_async_copy(
                    wcomm_ref.at[h, :, pl.ds(d * n2, n2)],
                    wstage_ref.at[d],
                    wcopy_sems.at[d],
                )
                cp.start()
                cps.append(cp)
            for d, origin in ((0, o_cw), (1, o_ccw)):
                slot = 2 * h + d
                cps[d].wait()
                for s in range(m_per // SUB):
                    yv = matmul(s * SUB, wstage_ref[d])
                    ysend_ref[slot, pl.ds(s * SUB, SUB), :] = yv
                    amax = jnp.maximum(amax, jnp.max(yv))
                r = pltpu.make_async_remote_copy(
                    src_ref=ysend_ref.at[slot],
                    dst_ref=out_ref.at[pl.ds(my * m_per, m_per),
                                       pl.ds(d * n2, n2)],
                    send_sem=y_send.at[slot],
                    recv_sem=y_recv.at[slot],
                    device_id=(origin,),
                    device_id_type=pl.DeviceIdType.MESH,
                )
                r.start()
                y_rdmas.append(r)
            return amax

        amax = jnp.float32(0)

        for h in range(N_DEV - 1):
            src = w_ref if h == 0 else wcomm_ref.at[h - 1]
            cw = pltpu.make_async_remote_copy(
                src_ref=src.at[:, pl.ds(0, n2)],
                dst_ref=wcomm_ref.at[h, :, pl.ds(0, n2)],
                send_sem=cw_send.at[h],
                recv_sem=cw_recv.at[h],
                device_id=(right,),
                device_id_type=pl.DeviceIdType.MESH,
            )
            ccw = pltpu.make_async_remote_copy(
                src_ref=src.at[:, pl.ds(n2, n2)],
                dst_ref=wcomm_ref.at[h, :, pl.ds(n2, n2)],
                send_sem=ccw_send.at[h],
                recv_sem=ccw_recv.at[h],
                device_id=(left,),
                device_id_type=pl.DeviceIdType.MESH,
            )
            cw.start()
            ccw.start()
            if h == 0:
                for s in range(m_per // SUB):
                    yv = matmul(s * SUB, w_ref[...])
                    out_ref[pl.ds(my * m_per + s * SUB, SUB), :] = yv
                    amax = jnp.maximum(amax, jnp.max(yv))
            else:
                amax = process_w_hop(h - 1, amax)
            cw.wait()
            ccw.wait()
        amax = process_w_hop(N_DEV - 2, amax)

        amax_ref[pl.ds(my, 1), :, :] = jnp.full((1, 8, 128), amax,
                                                jnp.float32)
        a_rdmas = []
        for o in range(1, N_DEV):
            tgt = lax.rem(my + o, N_DEV)
            r = pltpu.make_async_remote_copy(
                src_ref=amax_ref.at[pl.ds(my, 1)],
                dst_ref=amax_ref.at[pl.ds(my, 1)],
                send_sem=a_send.at[o - 1],
                recv_sem=a_recv.at[o - 1],
                device_id=(tgt,),
                device_id_type=pl.DeviceIdType.MESH,
            )
            r.start()
            a_rdmas.append(r)

        for r in y_rdmas:
            r.wait()
        for r in a_rdmas:
            r.wait()

        g_amax = jnp.max(amax_ref[...])
        scale = g_amax / 448.0
        inv_scale = 448.0 / g_amax
        for b in range(N_DEV):
            blk = out_ref[pl.ds(b * m_per, m_per), :]
            q = (blk * inv_scale).astype(jnp.float8_e4m3fn)
            out_ref[pl.ds(b * m_per, m_per), :] = (
                q.astype(jnp.float32) * scale)

    n_slots = 2 * (N_DEV - 1)
    out, _ = pl.pallas_call(
        body,
        out_shape=[
            jax.ShapeDtypeStruct((N_DEV * m_per, n_per), jnp.float32),
            jax.ShapeDtypeStruct((N_DEV - 1, k, n_per), jnp.float32),
        ],
        in_specs=[
            pl.BlockSpec(memory_space=pltpu.VMEM),
            pl.BlockSpec(memory_space=pltpu.VMEM),
        ],
        out_specs=[
            pl.BlockSpec(memory_space=pltpu.VMEM),
            pl.BlockSpec(memory_space=pltpu.HBM),
        ],
        scratch_shapes=[
            pltpu.VMEM((2, k, n2), jnp.float32),
            pltpu.VMEM((n_slots, m_per, n2), jnp.float32),
            pltpu.VMEM((N_DEV, 8, 128), jnp.float32),
            pltpu.SemaphoreType.DMA((N_DEV - 1,)),
            pltpu.SemaphoreType.DMA((N_DEV - 1,)),
            pltpu.SemaphoreType.DMA((N_DEV - 1,)),
            pltpu.SemaphoreType.DMA((N_DEV - 1,)),
            pltpu.SemaphoreType.DMA((n_slots,)),
            pltpu.SemaphoreType.DMA((n_slots,)),
            pltpu.SemaphoreType.DMA((N_DEV - 1,)),
            pltpu.SemaphoreType.DMA((N_DEV - 1,)),
            pltpu.SemaphoreType.DMA((2,)),
        ],
        compiler_params=pltpu.CompilerParams(collective_id=0),
    )(x, w_mat)
    return out


# device time: 181666 ns/iter; 3.9151x vs baseline; 1.1261x over previous
import jax
import jax.numpy as jnp
from jax import lax
from jax.experimental import pallas as pl
from jax.experimental.pallas import tpu as pltpu

N_DEV = 4
SUB = 256


def kernel(x, w_mat):
    m_per, k = x.shape
    _, n_per = w_mat.shape
    n2 = n_per // 2

    def body(x_ref, w_ref, out_ref, wcomm_ref, wstage_ref, ysend_ref,
             yrecv_ref, amax_ref, cw_send, cw_recv, ccw_send, ccw_recv,
             y_send, y_recv, a_send, a_recv, wcopy_sems):
        my = lax.axis_index("i")
        left = lax.rem(my + N_DEV - 1, N_DEV)
        right = lax.rem(my + 1, N_DEV)

        barrier_sem = pltpu.get_barrier_semaphore()
        for nbr in (left, right):
            pl.semaphore_signal(
                barrier_sem, inc=1,
                device_id=(nbr,), device_id_type=pl.DeviceIdType.MESH,
            )
        pl.semaphore_wait(barrier_sem, 2)

        def matmul(x_rows, w_val):
            y = lax.dot_general(
                x_ref[pl.ds(x_rows, SUB), :], w_val,
                dimension_numbers=(((1,), (0,)), ((), ())),
                precision=lax.Precision.DEFAULT,
                preferred_element_type=jnp.float32,
            )
            return jnp.maximum(y, 0.0)

        y_rdmas = []

        def process_w_hop(h, amax):
            o_cw = lax.rem(my + 2 * N_DEV - 1 - h, N_DEV)
            o_ccw = lax.rem(my + 1 + h, N_DEV)
            cps = []
            for d in range(2):
                cp = pltpu.make_async_copy(
                    wcomm_ref.at[h, :, pl.ds(d * n2, n2)],
                    wstage_ref.at[d],
                    wcopy_sems.at[d],
                )
                cp.start()
                cps.append(cp)
            for d, origin in ((0, o_cw), (1, o_ccw)):
                slot = 2 * h + d
                cps[d].wait()
                for s in range(m_per // SUB):
                    yv = matmul(s * SUB, wstage_ref[d])
                    ysend_ref[slot, pl.ds(s * SUB, SUB), :] = (
                        yv.astype(jnp.bfloat16))
                    amax = jnp.maximum(amax, jnp.max(yv))
                r = pltpu.make_async_remote_copy(
                    src_ref=ysend_ref.at[slot],
                    dst_ref=yrecv_ref.at[slot],
                    send_sem=y_send.at[slot],
                    recv_sem=y_recv.at[slot],
                    device_id=(origin,),
                    device_id_type=pl.DeviceIdType.MESH,
                )
                r.start()
                y_rdmas.append(r)
            return amax

        amax = jnp.float32(0)

        for h in range(N_DEV - 1):
            src = w_ref if h == 0 else wcomm_ref.at[h - 1]
            cw = pltpu.make_async_remote_copy(
                src_ref=src.at[:, pl.ds(0, n2)],
                dst_ref=wcomm_ref.at[h, :, pl.ds(0, n2)],
                send_sem=cw_send.at[h],
                recv_sem=cw_recv.at[h],
                device_id=(right,),
                device_id_type=pl.DeviceIdType.MESH,
            )
            ccw = pltpu.make_async_remote_copy(
                src_ref=src.at[:, pl.ds(n2, n2)],
                dst_ref=wcomm_ref.at[h, :, pl.ds(n2, n2)],
                send_sem=ccw_send.at[h],
                recv_sem=ccw_recv.at[h],
                device_id=(left,),
                device_id_type=pl.DeviceIdType.MESH,
            )
            cw.start()
            ccw.start()
            if h == 0:
                for s in range(m_per // SUB):
                    yv = matmul(s * SUB, w_ref[...])
                    out_ref[pl.ds(my * m_per + s * SUB, SUB), :] = yv
                    amax = jnp.maximum(amax, jnp.max(yv))
            else:
                amax = process_w_hop(h - 1, amax)
            cw.wait()
            ccw.wait()
        amax = process_w_hop(N_DEV - 2, amax)

        amax_ref[pl.ds(my, 1), :, :] = jnp.full((1, 8, 128), amax,
                                                jnp.float32)
        a_rdmas = []
        for o in range(1, N_DEV):
            tgt = lax.rem(my + o, N_DEV)
            r = pltpu.make_async_remote_copy(
                src_ref=amax_ref.at[pl.ds(my, 1)],
                dst_ref=amax_ref.at[pl.ds(my, 1)],
                send_sem=a_send.at[o - 1],
                recv_sem=a_recv.at[o - 1],
                device_id=(tgt,),
                device_id_type=pl.DeviceIdType.MESH,
            )
            r.start()
            a_rdmas.append(r)

        for r in y_rdmas:
            r.wait()
        for r in a_rdmas:
            r.wait()

        g_amax = jnp.max(amax_ref[...])
        scale = g_amax / 448.0
        inv_scale = 448.0 / g_amax

        def quant(vals_f32, rows, cols, width):
            q = jnp.minimum(vals_f32 * inv_scale, 448.0)
            q = q.astype(jnp.float8_e4m3fn)
            out_ref[pl.ds(rows, m_per), pl.ds(cols, width)] = (
                q.astype(jnp.float32) * scale)

        quant(out_ref[pl.ds(my * m_per, m_per), :], my * m_per, 0, n_per)
        for h in range(N_DEV - 1):
            for d, src_pos in ((0, lax.rem(my + 1 + h, N_DEV)),
                               (1, lax.rem(my + 2 * N_DEV - 1 - h, N_DEV))):
                quant(yrecv_ref[2 * h + d].astype(jnp.float32),
                      src_pos * m_per, d * n2, n2)

    n_slots = 2 * (N_DEV - 1)
    out, _ = pl.pallas_call(
        body,
        out_shape=[
            jax.ShapeDtypeStruct((N_DEV * m_per, n_per), jnp.float32),
            jax.ShapeDtypeStruct((N_DEV - 1, k, n_per), jnp.float32),
        ],
        in_specs=[
            pl.BlockSpec(memory_space=pltpu.VMEM),
            pl.BlockSpec(memory_space=pltpu.VMEM),
        ],
        out_specs=[
            pl.BlockSpec(memory_space=pltpu.VMEM),
            pl.BlockSpec(memory_space=pltpu.HBM),
        ],
        scratch_shapes=[
            pltpu.VMEM((2, k, n2), jnp.float32),
            pltpu.VMEM((n_slots, m_per, n2), jnp.bfloat16),
            pltpu.VMEM((n_slots, m_per, n2), jnp.bfloat16),
            pltpu.VMEM((N_DEV, 8, 128), jnp.float32),
            pltpu.SemaphoreType.DMA((N_DEV - 1,)),
            pltpu.SemaphoreType.DMA((N_DEV - 1,)),
            pltpu.SemaphoreType.DMA((N_DEV - 1,)),
            pltpu.SemaphoreType.DMA((N_DEV - 1,)),
            pltpu.SemaphoreType.DMA((n_slots,)),
            pltpu.SemaphoreType.DMA((n_slots,)),
            pltpu.SemaphoreType.DMA((N_DEV - 1,)),
            pltpu.SemaphoreType.DMA((N_DEV - 1,)),
            pltpu.SemaphoreType.DMA((2,)),
        ],
        compiler_params=pltpu.CompilerParams(collective_id=0),
    )(x, w_mat)
    return out


# device time: 179513 ns/iter; 3.9621x vs baseline; 1.0120x over previous
import jax
import jax.numpy as jnp
from jax import lax
from jax.experimental import pallas as pl
from jax.experimental.pallas import tpu as pltpu

N_DEV = 4
SUB = 256


def kernel(x, w_mat):
    m_per, k = x.shape
    _, n_per = w_mat.shape
    n2 = n_per // 2

    def body(x_ref, w_ref, out_ref, wcomm_ref, wstage_ref, ysend_ref,
             yrecv_ref, amax_ref, cw_send, cw_recv, ccw_send, ccw_recv,
             y_send, y_recv, a_send, a_recv, wcopy_sems):
        my = lax.axis_index("i")
        left = lax.rem(my + N_DEV - 1, N_DEV)
        right = lax.rem(my + 1, N_DEV)

        barrier_sem = pltpu.get_barrier_semaphore()
        for nbr in (left, right):
            pl.semaphore_signal(
                barrier_sem, inc=1,
                device_id=(nbr,), device_id_type=pl.DeviceIdType.MESH,
            )
        pl.semaphore_wait(barrier_sem, 2)

        def matmul(x_rows, w_val):
            y = lax.dot_general(
                x_ref[pl.ds(x_rows, SUB), :], w_val,
                dimension_numbers=(((1,), (0,)), ((), ())),
                precision=lax.Precision.DEFAULT,
                preferred_element_type=jnp.float32,
            )
            return jnp.maximum(y, 0.0)

        y_rdmas = {}

        def y_push(slot, origin):
            r = pltpu.make_async_remote_copy(
                src_ref=ysend_ref.at[slot],
                dst_ref=yrecv_ref.at[slot],
                send_sem=y_send.at[slot],
                recv_sem=y_recv.at[slot],
                device_id=(origin,),
                device_id_type=pl.DeviceIdType.MESH,
            )
            r.start()
            y_rdmas[slot] = r

        def process_w_hop(h, amax, defer_push=False):
            o_cw = lax.rem(my + 2 * N_DEV - 1 - h, N_DEV)
            o_ccw = lax.rem(my + 1 + h, N_DEV)
            cps = []
            for d in range(2):
                cp = pltpu.make_async_copy(
                    wcomm_ref.at[h, :, pl.ds(d * n2, n2)],
                    wstage_ref.at[d],
                    wcopy_sems.at[d],
                )
                cp.start()
                cps.append(cp)
            deferred = []
            for d, origin in ((0, o_cw), (1, o_ccw)):
                slot = 2 * h + d
                cps[d].wait()
                for s in range(m_per // SUB):
                    yv = matmul(s * SUB, wstage_ref[d])
                    ysend_ref[slot, pl.ds(s * SUB, SUB), :] = (
                        yv.astype(jnp.bfloat16))
                    amax = jnp.maximum(amax, jnp.max(yv))
                if defer_push:
                    deferred.append((slot, origin))
                else:
                    y_push(slot, origin)
            return amax, deferred

        amax = jnp.float32(0)

        for h in range(N_DEV - 1):
            src = w_ref if h == 0 else wcomm_ref.at[h - 1]
            cw = pltpu.make_async_remote_copy(
                src_ref=src.at[:, pl.ds(0, n2)],
                dst_ref=wcomm_ref.at[h, :, pl.ds(0, n2)],
                send_sem=cw_send.at[h],
                recv_sem=cw_recv.at[h],
                device_id=(right,),
                device_id_type=pl.DeviceIdType.MESH,
            )
            ccw = pltpu.make_async_remote_copy(
                src_ref=src.at[:, pl.ds(n2, n2)],
                dst_ref=wcomm_ref.at[h, :, pl.ds(n2, n2)],
                send_sem=ccw_send.at[h],
                recv_sem=ccw_recv.at[h],
                device_id=(left,),
                device_id_type=pl.DeviceIdType.MESH,
            )
            cw.start()
            ccw.start()
            if h == 0:
                for s in range(m_per // SUB):
                    yv = matmul(s * SUB, w_ref[...])
                    out_ref[pl.ds(my * m_per + s * SUB, SUB), :] = yv
                    amax = jnp.maximum(amax, jnp.max(yv))
            else:
                amax, _ = process_w_hop(h - 1, amax)
            cw.wait()
            ccw.wait()
        amax, deferred = process_w_hop(N_DEV - 2, amax, defer_push=True)

        amax_ref[pl.ds(my, 1), :, :] = jnp.full((1, 8, 128), amax,
                                                jnp.float32)
        a_rdmas = []
        for o in range(1, N_DEV):
            tgt = lax.rem(my + o, N_DEV)
            r = pltpu.make_async_remote_copy(
                src_ref=amax_ref.at[pl.ds(my, 1)],
                dst_ref=amax_ref.at[pl.ds(my, 1)],
                send_sem=a_send.at[o - 1],
                recv_sem=a_recv.at[o - 1],
                device_id=(tgt,),
                device_id_type=pl.DeviceIdType.MESH,
            )
            r.start()
            a_rdmas.append(r)

        for slot, origin in deferred:
            y_push(slot, origin)

        for r in a_rdmas:
            r.wait()

        g_amax = jnp.max(amax_ref[...])
        scale = g_amax / 448.0
        inv_scale = 448.0 / g_amax

        def quant(vals_f32, rows, cols, width):
            q = jnp.minimum(vals_f32 * inv_scale, 448.0)
            q = q.astype(jnp.float8_e4m3fn)
            out_ref[pl.ds(rows, m_per), pl.ds(cols, width)] = (
                q.astype(jnp.float32) * scale)

        quant(out_ref[pl.ds(my * m_per, m_per), :], my * m_per, 0, n_per)
        for h in range(N_DEV - 1):
            for d, src_pos in ((0, lax.rem(my + 1 + h, N_DEV)),
                               (1, lax.rem(my + 2 * N_DEV - 1 - h, N_DEV))):
                slot = 2 * h + d
                y_rdmas[slot].wait_recv()
                quant(yrecv_ref[slot].astype(jnp.float32),
                      src_pos * m_per, d * n2, n2)
        for slot in sorted(y_rdmas):
            y_rdmas[slot].wait_send()

    n_slots = 2 * (N_DEV - 1)
    out, _ = pl.pallas_call(
        body,
        out_shape=[
            jax.ShapeDtypeStruct((N_DEV * m_per, n_per), jnp.float32),
            jax.ShapeDtypeStruct((N_DEV - 1, k, n_per), jnp.float32),
        ],
        in_specs=[
            pl.BlockSpec(memory_space=pltpu.VMEM),
            pl.BlockSpec(memory_space=pltpu.VMEM),
        ],
        out_specs=[
            pl.BlockSpec(memory_space=pltpu.VMEM),
            pl.BlockSpec(memory_space=pltpu.HBM),
        ],
        scratch_shapes=[
            pltpu.VMEM((2, k, n2), jnp.float32),
            pltpu.VMEM((n_slots, m_per, n2), jnp.bfloat16),
            pltpu.VMEM((n_slots, m_per, n2), jnp.bfloat16),
            pltpu.VMEM((N_DEV, 8, 128), jnp.float32),
            pltpu.SemaphoreType.DMA((N_DEV - 1,)),
            pltpu.SemaphoreType.DMA((N_DEV - 1,)),
            pltpu.SemaphoreType.DMA((N_DEV - 1,)),
            pltpu.SemaphoreType.DMA((N_DEV - 1,)),
            pltpu.SemaphoreType.DMA((n_slots,)),
            pltpu.SemaphoreType.DMA((n_slots,)),
            pltpu.SemaphoreType.DMA((N_DEV - 1,)),
            pltpu.SemaphoreType.DMA((N_DEV - 1,)),
            pltpu.SemaphoreType.DMA((2,)),
        ],
        compiler_params=pltpu.CompilerParams(collective_id=0),
    )(x, w_mat)
    return out


# device time: 112404 ns/iter; 6.3276x vs baseline; 1.5970x over previous
import jax
import jax.numpy as jnp
from jax import lax
from jax.experimental import pallas as pl
from jax.experimental.pallas import tpu as pltpu

N_DEV = 4
SUB = 256


def kernel(x, w_mat):
    m_per, k = x.shape
    _, n_per = w_mat.shape
    n2 = n_per // 2

    def body(x_ref, w_ref, out_ref, wcomm_ref, wbf_ref, wstage_ref,
             ysend_ref, yrecv_ref, amax_ref,
             cw_send, cw_recv, ccw_send, ccw_recv,
             y_send, y_recv, a_send, a_recv, wcopy_sems):
        my = lax.axis_index("i")
        left = lax.rem(my + N_DEV - 1, N_DEV)
        right = lax.rem(my + 1, N_DEV)

        wbf_ref[...] = w_ref[...].astype(jnp.bfloat16)

        barrier_sem = pltpu.get_barrier_semaphore()
        for nbr in (left, right):
            pl.semaphore_signal(
                barrier_sem, inc=1,
                device_id=(nbr,), device_id_type=pl.DeviceIdType.MESH,
            )
        pl.semaphore_wait(barrier_sem, 2)

        def matmul(x_rows, w_val):
            y = lax.dot_general(
                x_ref[pl.ds(x_rows, SUB), :], w_val,
                dimension_numbers=(((1,), (0,)), ((), ())),
                precision=lax.Precision.DEFAULT,
                preferred_element_type=jnp.float32,
            )
            return jnp.maximum(y, 0.0)

        y_rdmas = {}

        def y_push(slot, origin):
            r = pltpu.make_async_remote_copy(
                src_ref=ysend_ref.at[slot],
                dst_ref=yrecv_ref.at[slot],
                send_sem=y_send.at[slot],
                recv_sem=y_recv.at[slot],
                device_id=(origin,),
                device_id_type=pl.DeviceIdType.MESH,
            )
            r.start()
            y_rdmas[slot] = r

        def process_w_hop(h, amax, defer_push=False):
            o_cw = lax.rem(my + 2 * N_DEV - 1 - h, N_DEV)
            o_ccw = lax.rem(my + 1 + h, N_DEV)
            cps = []
            for d in range(2):
                cp = pltpu.make_async_copy(
                    wcomm_ref.at[h, :, pl.ds(d * n2, n2)],
                    wstage_ref.at[d],
                    wcopy_sems.at[d],
                )
                cp.start()
                cps.append(cp)
            deferred = []
            for d, origin in ((0, o_cw), (1, o_ccw)):
                slot = 2 * h + d
                cps[d].wait()
                for s in range(m_per // SUB):
                    yv = matmul(s * SUB, wstage_ref[d])
                    ysend_ref[slot, pl.ds(s * SUB, SUB), :] = (
                        yv.astype(jnp.bfloat16))
                    amax = jnp.maximum(amax, jnp.max(yv))
                if defer_push:
                    deferred.append((slot, origin))
                else:
                    y_push(slot, origin)
            return amax, deferred

        amax = jnp.float32(0)

        for h in range(N_DEV - 1):
            src = wbf_ref if h == 0 else wcomm_ref.at[h - 1]
            cw = pltpu.make_async_remote_copy(
                src_ref=src.at[:, pl.ds(0, n2)],
                dst_ref=wcomm_ref.at[h, :, pl.ds(0, n2)],
                send_sem=cw_send.at[h],
                recv_sem=cw_recv.at[h],
                device_id=(right,),
                device_id_type=pl.DeviceIdType.MESH,
            )
            ccw = pltpu.make_async_remote_copy(
                src_ref=src.at[:, pl.ds(n2, n2)],
                dst_ref=wcomm_ref.at[h, :, pl.ds(n2, n2)],
                send_sem=ccw_send.at[h],
                recv_sem=ccw_recv.at[h],
                device_id=(left,),
                device_id_type=pl.DeviceIdType.MESH,
            )
            cw.start()
            ccw.start()
            if h == 0:
                for s in range(m_per // SUB):
                    yv = matmul(s * SUB, w_ref[...])
                    out_ref[pl.ds(my * m_per + s * SUB, SUB), :] = yv
                    amax = jnp.maximum(amax, jnp.max(yv))
            else:
                amax, _ = process_w_hop(h - 1, amax)
            cw.wait()
            ccw.wait()
        amax, deferred = process_w_hop(N_DEV - 2, amax, defer_push=True)

        amax_ref[pl.ds(my, 1), :, :] = jnp.full((1, 8, 128), amax,
                                                jnp.float32)
        a_rdmas = []
        for o in range(1, N_DEV):
            tgt = lax.rem(my + o, N_DEV)
            r = pltpu.make_async_remote_copy(
                src_ref=amax_ref.at[pl.ds(my, 1)],
                dst_ref=amax_ref.at[pl.ds(my, 1)],
                send_sem=a_send.at[o - 1],
                recv_sem=a_recv.at[o - 1],
                device_id=(tgt,),
                device_id_type=pl.DeviceIdType.MESH,
            )
            r.start()
            a_rdmas.append(r)

        for slot, origin in deferred:
            y_push(slot, origin)

        for r in a_rdmas:
            r.wait()

        g_amax = jnp.max(amax_ref[...])
        scale = g_amax / 448.0
        inv_scale = 448.0 / g_amax

        def quant(vals_f32, rows, cols, width):
            q = jnp.minimum(vals_f32 * inv_scale, 448.0)
            q = q.astype(jnp.float8_e4m3fn)
            out_ref[pl.ds(rows, m_per), pl.ds(cols, width)] = (
                q.astype(jnp.float32) * scale)

        quant(out_ref[pl.ds(my * m_per, m_per), :], my * m_per, 0, n_per)
        for h in range(N_DEV - 1):
            for d, src_pos in ((0, lax.rem(my + 1 + h, N_DEV)),
                               (1, lax.rem(my + 2 * N_DEV - 1 - h, N_DEV))):
                slot = 2 * h + d
                y_rdmas[slot].wait_recv()
                quant(yrecv_ref[slot].astype(jnp.float32),
                      src_pos * m_per, d * n2, n2)
        for slot in sorted(y_rdmas):
            y_rdmas[slot].wait_send()

    n_slots = 2 * (N_DEV - 1)
    out, _ = pl.pallas_call(
        body,
        out_shape=[
            jax.ShapeDtypeStruct((N_DEV * m_per, n_per), jnp.float32),
            jax.ShapeDtypeStruct((N_DEV - 1, k, n_per), jnp.bfloat16),
        ],
        in_specs=[
            pl.BlockSpec(memory_space=pltpu.VMEM),
            pl.BlockSpec(memory_space=pltpu.VMEM),
        ],
        out_specs=[
            pl.BlockSpec(memory_space=pltpu.VMEM),
            pl.BlockSpec(memory_space=pltpu.HBM),
        ],
        scratch_shapes=[
            pltpu.VMEM((k, n_per), jnp.bfloat16),
            pltpu.VMEM((2, k, n2), jnp.bfloat16),
            pltpu.VMEM((n_slots, m_per, n2), jnp.bfloat16),
            pltpu.VMEM((n_slots, m_per, n2), jnp.bfloat16),
            pltpu.VMEM((N_DEV, 8, 128), jnp.float32),
            pltpu.SemaphoreType.DMA((N_DEV - 1,)),
            pltpu.SemaphoreType.DMA((N_DEV - 1,)),
            pltpu.SemaphoreType.DMA((N_DEV - 1,)),
            pltpu.SemaphoreType.DMA((N_DEV - 1,)),
            pltpu.SemaphoreType.DMA((n_slots,)),
            pltpu.SemaphoreType.DMA((n_slots,)),
            pltpu.SemaphoreType.DMA((N_DEV - 1,)),
            pltpu.SemaphoreType.DMA((N_DEV - 1,)),
            pltpu.SemaphoreType.DMA((2,)),
        ],
        compiler_params=pltpu.CompilerParams(collective_id=0),
    )(x, w_mat)
    return out
